# baseline (speedup 1.0000x reference)
import functools

import jax
import jax.numpy as jnp
from jax import lax
from jax.experimental import pallas as pl
from jax.experimental.pallas import tpu as pltpu
from jax.experimental.pallas import tpu_sc as plsc

_NFIELD = 26
_FIELD_SIZE = 100000
_BATCH = 4096
_D = 64
_BF = _BATCH * _NFIELD
_NW = 32
_BPW = _BF // _NW
_CHUNK = 128
_NCHUNK = _BPW // _CHUNK
_NBUF = 4
_L = 16


def _body(x_hbm, table_hbm, out_hbm, idx_v, rows_v, *sems):
    gsems = sems[:_NBUF]
    wsems = sems[_NBUF:]
    wid = lax.axis_index("s") * 2 + lax.axis_index("c")
    base = wid * _BPW

    pltpu.sync_copy(x_hbm.at[pl.ds(base, _BPW)], idx_v)

    def _add_offsets(i, carry):
        pos = (base + i * _L) + lax.iota(jnp.int32, _L)
        off = (pos % _NFIELD) * _FIELD_SIZE
        idx_v[pl.ds(i * _L, _L)] = idx_v[pl.ds(i * _L, _L)] + off
        return carry

    lax.fori_loop(0, _BPW // _L, _add_offsets, 0)
    plsc.subcore_barrier()

    def _gather(c, b):
        return pltpu.async_copy(
            table_hbm.at[idx_v.at[pl.ds(c * _CHUNK, _CHUNK)]],
            rows_v.at[b],
            gsems[b],
        )

    def _write(c, b):
        return pltpu.async_copy(
            rows_v.at[b],
            out_hbm.at[pl.ds(base + c * _CHUNK, _CHUNK)],
            wsems[b],
        )

    g = {}
    w = {}
    for c in range(min(_NBUF, _NCHUNK)):
        g[c] = _gather(c, c)
    for c in range(_NCHUNK):
        b = c % _NBUF
        g[c].wait()
        w[c] = _write(c, b)
        n = c + _NBUF
        if n < _NCHUNK:
            w[c].wait()
            g[n] = _gather(n, b)
    for c in range(max(0, _NCHUNK - _NBUF), _NCHUNK):
        w[c].wait()


@functools.cache
def _sc_gather():
    mesh = plsc.VectorSubcoreMesh(core_axis_name="c", subcore_axis_name="s")
    return functools.partial(
        pl.kernel,
        out_type=jax.ShapeDtypeStruct((_BF, _D), jnp.float32),
        scratch_types=[
            pltpu.VMEM((_BPW,), jnp.int32),
            pltpu.VMEM((_NBUF, _CHUNK, _D), jnp.float32),
        ]
        + [pltpu.SemaphoreType.DMA] * (2 * _NBUF),
        mesh=mesh,
        compiler_params=pltpu.CompilerParams(use_tc_tiling_on_sc=False),
    )(_body)


@jax.jit
def kernel(x, table):
    xf = x.reshape(-1).astype(jnp.int32)
    out = _sc_gather()(xf, table)
    return out.reshape(_BATCH, _NFIELD, _D)
